# Initial kernel scaffold; baseline (speedup 1.0000x reference)
import jax
import jax.numpy as jnp
from jax.experimental import pallas as pl


def kernel(left_input, right_input):
    raise NotImplementedError("write your pallas kernel here")


# speedup vs baseline: 1.0027x; 1.0027x over previous
import jax
import jax.numpy as jnp
from jax.experimental import pallas as pl


def _mul_body(l_ref, r_ref, o_ref):
    o_ref[...] = l_ref[...] * r_ref[...]


def kernel(left_input, right_input):
    B, F = left_input.shape
    ROWS = 2048
    return pl.pallas_call(
        _mul_body,
        grid=(B // ROWS,),
        in_specs=[
            pl.BlockSpec((ROWS, F), lambda i: (i, 0)),
            pl.BlockSpec((ROWS, F), lambda i: (i, 0)),
        ],
        out_specs=pl.BlockSpec((ROWS, F), lambda i: (i, 0)),
        out_shape=jax.ShapeDtypeStruct((B, F), left_input.dtype),
    )(left_input, right_input)
